# Initial kernel scaffold; baseline (speedup 1.0000x reference)
import jax
import jax.numpy as jnp
from jax.experimental import pallas as pl


def kernel(t_float, X_t_one_hot, edge_index, edge_weight, t_W1, t_b1, t_W2, t_b2, x_W1, x_b1, x_W2, x_b2, g_W0, g_b0, ln_g0, ln_b0, g_W1, g_b1, ln_g1, ln_b1, g_W2, g_b2, ln_g2, ln_b2, o_W1, o_b1, o_W2, o_b2):
    raise NotImplementedError("write your pallas kernel here")


# speedup vs baseline: 2.6685x; 2.6685x over previous
import functools

import jax
import jax.numpy as jnp
from jax import lax
from jax.experimental import pallas as pl
from jax.experimental.pallas import tpu as pltpu
from jax.experimental.pallas import tpu_sc as plsc

_N = 10000
_HX = 256
_HH = 128
_HT = 128
_HCAT = 4 * _HX + _HT
_NC = 2
_NS = 16
_C = 128
_R = 400


def _prelude_body(t_ref, tw1, tb1, tw2, tb2, gt0, gb0, gt1, gb1, gt2, gb2,
                  owt, ob1, bg0, bg1, bg2, bo):
    x1 = jnp.maximum(t_ref[...] * tw1[...] + tb1[...], 0.0)
    ht = jnp.maximum(
        jnp.dot(x1, tw2[...], preferred_element_type=jnp.float32) + tb2[...],
        0.0)
    bg0[...] = jnp.dot(ht, gt0[...], preferred_element_type=jnp.float32) + gb0[...]
    bg1[...] = jnp.dot(ht, gt1[...], preferred_element_type=jnp.float32) + gb1[...]
    bg2[...] = jnp.dot(ht, gt2[...], preferred_element_type=jnp.float32) + gb2[...]
    bo[...] = jnp.dot(ht, owt[...], preferred_element_type=jnp.float32) + ob1[...]


def _xmlp_body(x, w1, b1, w2, b2, out):
    h1 = jnp.maximum(
        jnp.dot(x[...], w1[...], preferred_element_type=jnp.float32) + b1[...], 0.0)
    h = jnp.maximum(
        jnp.dot(h1, w2[...], preferred_element_type=jnp.float32) + b2[...], 0.0)
    out[0] = h[:, :_HH]
    out[1] = h[:, _HH:]


def _layer_body(a, w, bias, g, b, out):
    y = (jnp.dot(a[0], w[0], preferred_element_type=jnp.float32)
         + jnp.dot(a[1], w[1], preferred_element_type=jnp.float32)
         + bias[...])
    y = jnp.maximum(y, 0.0)
    m = jnp.mean(y, axis=-1, keepdims=True)
    yc = y - m
    v = jnp.mean(yc * yc, axis=-1, keepdims=True)
    h = yc * lax.rsqrt(v + 1e-5) * g[...] + b[...]
    out[0] = h[:, :_HH]
    out[1] = h[:, _HH:]


def _out_body(h0, h1, h2, h3, w1r, bo, w2, b2, out):
    s = None
    for k, h in enumerate((h0, h1, h2, h3)):
        for cc in range(2):
            contrib = jnp.dot(h[cc], w1r[2 * k + cc],
                              preferred_element_type=jnp.float32)
            s = contrib if s is None else s + contrib
    y = jnp.maximum(s + bo[...], 0.0)
    out[...] = jnp.dot(y, w2[...], preferred_element_type=jnp.float32) + b2[...]


_K = 16


@functools.lru_cache(maxsize=None)
def _make_segsum(epad):
    ept = epad // _NS
    nch = ept // _C
    nst = nch // _K
    nfull = _N // _C
    ntail = _N - nfull * _C

    mesh = plsc.VectorSubcoreMesh(core_axis_name="c", subcore_axis_name="s",
                                  num_cores=_NC, num_subcores=_NS)

    @functools.partial(
        pl.kernel,
        out_type=jax.ShapeDtypeStruct((_NC * _N, _HH), jnp.float32),
        mesh=mesh,
        scratch_types=[
            pltpu.VMEM((_K, _C), jnp.int32),
            pltpu.VMEM((_K, _C), jnp.int32),
            pltpu.VMEM((_K * _C,), jnp.float32),
            pltpu.VMEM((_C, _HH), jnp.float32),
            pltpu.VMEM_SHARED((_N, _HH), jnp.float32),
            pltpu.SemaphoreType.DMA,
        ],
    )
    def segsum(hx, src2, dstr, wr, out, sidx, didx, wv, rows, aggr, sem):
        c = lax.axis_index("c")
        s = lax.axis_index("s")
        z = jnp.zeros((16,), jnp.float32)

        @pl.loop(0, _C)
        def _(r):
            for j in range(_HH // 16):
                rows[r, pl.ds(16 * j, 16)] = z

        @pl.loop(s, nfull, step=_NS)
        def _(k):
            pltpu.sync_copy(rows, aggr.at[pl.ds(k * _C, _C)])

        @pl.when(s == 0)
        def _():
            pltpu.sync_copy(rows.at[pl.ds(0, ntail)],
                            aggr.at[pl.ds(nfull * _C, ntail)])

        plsc.subcore_barrier()

        @pl.loop(0, nst)
        def _(st):
            pltpu.sync_copy(src2.at[c * _NS + s, st], sidx)
            pltpu.sync_copy(dstr.at[s, st], didx)
            pltpu.sync_copy(wr.at[s, st], wv)

            @pl.loop(0, _K)
            def _(ch):
                pltpu.async_copy(hx.at[sidx.at[ch]], rows, sem).wait()

                @pl.loop(0, _C // 16)
                def _(g):
                    wvec = wv[pl.ds(ch * _C + g * 16, 16)]
                    for ii in range(16):
                        wb = jnp.broadcast_to(wvec[ii], (16,))
                        i = g * 16 + ii
                        for j in range(_HH // 16):
                            rows[i, pl.ds(16 * j, 16)] = (
                                rows[i, pl.ds(16 * j, 16)] * wb)

                pltpu.sync_copy(rows, aggr.at[didx.at[ch]], add=True)

        plsc.subcore_barrier()

        @pl.loop(s, nfull, step=_NS)
        def _(k):
            pltpu.sync_copy(aggr.at[pl.ds(k * _C, _C)],
                            out.at[pl.ds(c * _N + k * _C, _C)])

        @pl.when(s == 0)
        def _():
            pltpu.sync_copy(aggr.at[pl.ds(nfull * _C, ntail)],
                            out.at[pl.ds(c * _N + nfull * _C, ntail)])

    return segsum


def kernel(t_float, X_t_one_hot, edge_index, edge_weight, t_W1, t_b1, t_W2,
           t_b2, x_W1, x_b1, x_W2, x_b2, g_W0, g_b0, ln_g0, ln_b0, g_W1, g_b1,
           ln_g1, ln_b1, g_W2, g_b2, ln_g2, ln_b2, o_W1, o_b1, o_W2, o_b2):
    E = edge_index.shape[1]
    epad = -(-E // (_NS * _C * _K)) * (_NS * _C * _K)
    pad = epad - E
    dst = edge_index[0]
    src = edge_index[1]
    srcp = jnp.pad(src, (0, pad))
    dstp = jnp.pad(dst, (0, pad))
    wp = jnp.pad(edge_weight, (0, pad))
    src2 = jnp.stack([srcp, srcp + _N]).reshape(_NC * _NS, -1, _K, _C)
    dstr = dstp.reshape(_NS, -1, _K, _C)
    wr = wp.reshape(_NS, -1, _K * _C)

    gws = (g_W0, g_W1, g_W2)
    gbs = (g_b0, g_b1, g_b2)
    lgs = (ln_g0, ln_g1, ln_g2)
    lbs = (ln_b0, ln_b1, ln_b2)

    vec = lambda v: v.reshape(1, -1)
    bg0, bg1, bg2, bo = pl.pallas_call(
        _prelude_body,
        out_shape=[jax.ShapeDtypeStruct((1, _HX), jnp.float32)] * 3
        + [jax.ShapeDtypeStruct((1, _HCAT), jnp.float32)],
    )(vec(t_float), t_W1, vec(t_b1), t_W2, vec(t_b2),
      gws[0][_HX:], vec(gbs[0]), gws[1][_HX:], vec(gbs[1]),
      gws[2][_HX:], vec(gbs[2]), o_W1[4 * _HX:], vec(o_b1))
    bgs = (bg0, bg1, bg2)

    grid = (_N // _R,)
    full2 = lambda shape: pl.BlockSpec(shape, lambda i: (0, 0))
    full3 = lambda shape: pl.BlockSpec(shape, lambda i: (0, 0, 0))
    hblk = pl.BlockSpec((2, _R, _HH), lambda i: (0, i, 0))

    h = pl.pallas_call(
        _xmlp_body,
        grid=grid,
        in_specs=[pl.BlockSpec((_R, 128), lambda i: (i, 0)),
                  full2((128, _HX)), full2((1, _HX)),
                  full2((_HX, _HX)), full2((1, _HX))],
        out_specs=hblk,
        out_shape=jax.ShapeDtypeStruct((2, _N, _HH), jnp.float32),
    )(X_t_one_hot, x_W1, vec(x_b1), x_W2, vec(x_b2))

    segsum = _make_segsum(epad)
    hs = [h]
    for l in range(3):
        aggr = segsum(h.reshape(_NC * _N, _HH), src2, dstr, wr)
        aggr = aggr.reshape(_NC, _N, _HH)
        h = pl.pallas_call(
            _layer_body,
            grid=grid,
            in_specs=[hblk, full3((2, _HH, _HX)), full2((1, _HX)),
                      full2((1, _HX)), full2((1, _HX))],
            out_specs=hblk,
            out_shape=jax.ShapeDtypeStruct((2, _N, _HH), jnp.float32),
        )(aggr, gws[l][:_HX].reshape(2, _HH, _HX), bgs[l],
          vec(lgs[l]), vec(lbs[l]))
        hs.append(h)

    out = pl.pallas_call(
        _out_body,
        grid=grid,
        in_specs=[hblk] * 4
        + [full3((8, _HH, _HCAT)), full2((1, _HCAT)),
           full2((_HCAT, 128)), full2((1, 128))],
        out_specs=pl.BlockSpec((_R, 128), lambda i: (i, 0)),
        out_shape=jax.ShapeDtypeStruct((_N, 128), jnp.float32),
    )(hs[0], hs[1], hs[2], hs[3],
      o_W1[:4 * _HX].reshape(8, _HH, _HCAT), bo, o_W2, vec(o_b2))
    return out


# speedup vs baseline: 5.5780x; 2.0903x over previous
import functools

import jax
import jax.numpy as jnp
from jax import lax
from jax.experimental import pallas as pl
from jax.experimental.pallas import tpu as pltpu
from jax.experimental.pallas import tpu_sc as plsc

_N = 10000
_HX = 256
_HH = 128
_HT = 128
_HCAT = 4 * _HX + _HT
_NC = 2
_NS = 16
_C = 112
_R = 400


def _prelude_body(t_ref, tw1, tb1, tw2, tb2, gt0, gb0, gt1, gb1, gt2, gb2,
                  owt, ob1, bg0, bg1, bg2, bo):
    x1 = jnp.maximum(t_ref[...] * tw1[...] + tb1[...], 0.0)
    ht = jnp.maximum(
        jnp.dot(x1, tw2[...], preferred_element_type=jnp.float32) + tb2[...],
        0.0)
    bg0[...] = jnp.dot(ht, gt0[...], preferred_element_type=jnp.float32) + gb0[...]
    bg1[...] = jnp.dot(ht, gt1[...], preferred_element_type=jnp.float32) + gb1[...]
    bg2[...] = jnp.dot(ht, gt2[...], preferred_element_type=jnp.float32) + gb2[...]
    bo[...] = jnp.dot(ht, owt[...], preferred_element_type=jnp.float32) + ob1[...]


def _xmlp_body(x, w1, b1, w2, b2, out):
    h1 = jnp.maximum(
        jnp.dot(x[...], w1[...], preferred_element_type=jnp.float32) + b1[...], 0.0)
    h = jnp.maximum(
        jnp.dot(h1, w2[...], preferred_element_type=jnp.float32) + b2[...], 0.0)
    out[0] = h[:, :_HH]
    out[1] = h[:, _HH:]


def _layer_body(a, w, bias, g, b, out):
    y = (jnp.dot(a[0], w[0], preferred_element_type=jnp.float32)
         + jnp.dot(a[1], w[1], preferred_element_type=jnp.float32)
         + bias[...])
    y = jnp.maximum(y, 0.0)
    m = jnp.mean(y, axis=-1, keepdims=True)
    yc = y - m
    v = jnp.mean(yc * yc, axis=-1, keepdims=True)
    h = yc * lax.rsqrt(v + 1e-5) * g[...] + b[...]
    out[0] = h[:, :_HH]
    out[1] = h[:, _HH:]


def _out_body(h0, h1, h2, h3, w1r, bo, w2, b2, out):
    s = None
    for k, h in enumerate((h0, h1, h2, h3)):
        for cc in range(2):
            contrib = jnp.dot(h[cc], w1r[2 * k + cc],
                              preferred_element_type=jnp.float32)
            s = contrib if s is None else s + contrib
    y = jnp.maximum(s + bo[...], 0.0)
    out[...] = jnp.dot(y, w2[...], preferred_element_type=jnp.float32) + b2[...]


_K = 6


@functools.lru_cache(maxsize=None)
def _make_segsum(epad):
    ept = epad // _NS
    nch = ept // _C
    nst = nch // _K
    assert nst % 2 == 0 and nst * _K * _NS * _C == epad
    nfull = _N // _C
    ntail = _N - nfull * _C

    mesh = plsc.VectorSubcoreMesh(core_axis_name="c", subcore_axis_name="s",
                                  num_cores=_NC, num_subcores=_NS)

    @functools.partial(
        pl.kernel,
        out_type=jax.ShapeDtypeStruct((_NC * _N, _HH), jnp.float32),
        mesh=mesh,
        scratch_types=[
            pltpu.VMEM((2, _K, _C), jnp.int32),
            pltpu.VMEM((2, _K, _C), jnp.int32),
            pltpu.VMEM((2, _K * _C), jnp.float32),
            pltpu.VMEM((_C, _HH), jnp.float32),
            pltpu.VMEM((_C, _HH), jnp.float32),
            pltpu.VMEM((_C, _HH), jnp.float32),
            pltpu.VMEM_SHARED((_N, _HH), jnp.float32),
            pltpu.SemaphoreType.DMA,
            pltpu.SemaphoreType.DMA,
            pltpu.SemaphoreType.DMA,
            pltpu.SemaphoreType.DMA,
            pltpu.SemaphoreType.DMA,
            pltpu.SemaphoreType.DMA,
            pltpu.SemaphoreType.DMA,
        ],
    )
    def segsum(hx, src2, dstr, wr, out, sidx, didx, wv, r0, r1, r2, aggr,
               g0, g1, g2, s0, s1, s2, isem):
        c = lax.axis_index("c")
        s = lax.axis_index("s")
        rowsl = (r0, r1, r2)
        gsems = (g0, g1, g2)
        ssems = (s0, s1, s2)
        widx = c * _NS + s
        z = jnp.zeros((16,), jnp.float32)

        @pl.loop(0, _C)
        def _(r):
            for j in range(_HH // 16):
                r0[r, pl.ds(16 * j, 16)] = z

        @pl.loop(s, nfull, step=_NS)
        def _(k):
            pltpu.sync_copy(r0, aggr.at[pl.ds(k * _C, _C)])

        @pl.when(s == 0)
        def _():
            pltpu.sync_copy(r0.at[pl.ds(0, ntail)],
                            aggr.at[pl.ds(nfull * _C, ntail)])

        plsc.subcore_barrier()

        def start_idx(pn, stn):
            pltpu.async_copy(src2.at[widx, stn], sidx.at[pn], isem)
            pltpu.async_copy(dstr.at[s, stn], didx.at[pn], isem)
            pltpu.async_copy(wr.at[s, stn], wv.at[pn], isem)

        def wait_idx(pn):
            pltpu.make_async_copy(src2.at[widx, 0], sidx.at[pn], isem).wait()
            pltpu.make_async_copy(dstr.at[s, 0], didx.at[pn], isem).wait()
            pltpu.make_async_copy(wr.at[s, 0], wv.at[pn], isem).wait()

        def start_gather(pn, ch, b):
            pltpu.async_copy(hx.at[sidx.at[pn, ch]], rowsl[b], gsems[b])

        def wait_gather(pn, ch, b):
            pltpu.make_async_copy(hx.at[sidx.at[pn, ch]], rowsl[b],
                                  gsems[b]).wait()

        def start_scatter(pn, ch, b):
            pltpu.sync_copy(rowsl[b], aggr.at[didx.at[pn, ch]], add=True)

        def wait_scatter(b):
            pass

        pltpu.sync_copy(src2.at[widx, 0], sidx.at[0])
        pltpu.sync_copy(dstr.at[s, 0], didx.at[0])
        pltpu.sync_copy(wr.at[s, 0], wv.at[0])
        start_gather(0, 0, 0)
        start_gather(0, 1, 1)

        @pl.loop(0, nst, step=2)
        def _(st0):
            for p in range(2):
                for ch in range(_K):
                    b = ch % 3
                    b2 = (b + 2) % 3
                    wait_gather(p, ch, b)
                    rows_b = rowsl[b]

                    @pl.loop(0, _C // 16)
                    def _(gg, p=p, ch=ch, rows_b=rows_b):
                        wvec = wv[p, pl.ds(ch * _C + gg * 16, 16)]
                        for ii in range(16):
                            wb = jnp.broadcast_to(wvec[ii], (16,))
                            for j in range(_HH // 16):
                                rows_b[gg * 16 + ii, pl.ds(16 * j, 16)] = (
                                    rows_b[gg * 16 + ii, pl.ds(16 * j, 16)]
                                    * wb)

                    if p == 0 and ch == 0:
                        @pl.when(st0 >= 1)
                        def _():
                            wait_scatter(b2)
                    else:
                        wait_scatter(b2)
                    if ch == 1:
                        if p == 0:
                            start_idx(1, st0 + 1)
                        else:
                            @pl.when(st0 + 2 < nst)
                            def _():
                                start_idx(0, st0 + 2)
                    if ch < _K - 2:
                        start_gather(p, ch + 2, b2)
                    elif ch == _K - 2:
                        if p == 0:
                            wait_idx(1)
                            start_gather(1, 0, b2)
                        else:
                            @pl.when(st0 + 2 < nst)
                            def _():
                                wait_idx(0)
                                start_gather(0, 0, b2)
                    else:
                        if p == 0:
                            start_gather(1, 1, b2)
                        else:
                            @pl.when(st0 + 2 < nst)
                            def _():
                                start_gather(0, 1, b2)
                    start_scatter(p, ch, b)

        wait_scatter((nst * _K - 1) % 3)

        plsc.subcore_barrier()

        @pl.loop(s, nfull, step=_NS)
        def _(k):
            pltpu.sync_copy(aggr.at[pl.ds(k * _C, _C)],
                            out.at[pl.ds(c * _N + k * _C, _C)])

        @pl.when(s == 0)
        def _():
            pltpu.sync_copy(aggr.at[pl.ds(nfull * _C, ntail)],
                            out.at[pl.ds(c * _N + nfull * _C, ntail)])

    return segsum


def kernel(t_float, X_t_one_hot, edge_index, edge_weight, t_W1, t_b1, t_W2,
           t_b2, x_W1, x_b1, x_W2, x_b2, g_W0, g_b0, ln_g0, ln_b0, g_W1, g_b1,
           ln_g1, ln_b1, g_W2, g_b2, ln_g2, ln_b2, o_W1, o_b1, o_W2, o_b2):
    E = edge_index.shape[1]
    nst = -(-E // (_NS * _C * _K))
    nst += nst % 2
    epad = nst * _NS * _C * _K
    pad = epad - E
    dst = edge_index[0]
    src = edge_index[1]
    srcp = jnp.pad(src, (0, pad))
    dstp = jnp.pad(dst, (0, pad))
    wp = jnp.pad(edge_weight, (0, pad))
    src2 = jnp.stack([srcp, srcp + _N]).reshape(_NC * _NS, -1, _K, _C)
    dstr = dstp.reshape(_NS, -1, _K, _C)
    wr = wp.reshape(_NS, -1, _K * _C)

    gws = (g_W0, g_W1, g_W2)
    gbs = (g_b0, g_b1, g_b2)
    lgs = (ln_g0, ln_g1, ln_g2)
    lbs = (ln_b0, ln_b1, ln_b2)

    vec = lambda v: v.reshape(1, -1)
    bg0, bg1, bg2, bo = pl.pallas_call(
        _prelude_body,
        out_shape=[jax.ShapeDtypeStruct((1, _HX), jnp.float32)] * 3
        + [jax.ShapeDtypeStruct((1, _HCAT), jnp.float32)],
    )(vec(t_float), t_W1, vec(t_b1), t_W2, vec(t_b2),
      gws[0][_HX:], vec(gbs[0]), gws[1][_HX:], vec(gbs[1]),
      gws[2][_HX:], vec(gbs[2]), o_W1[4 * _HX:], vec(o_b1))
    bgs = (bg0, bg1, bg2)

    grid = (_N // _R,)
    full2 = lambda shape: pl.BlockSpec(shape, lambda i: (0, 0))
    full3 = lambda shape: pl.BlockSpec(shape, lambda i: (0, 0, 0))
    hblk = pl.BlockSpec((2, _R, _HH), lambda i: (0, i, 0))

    h = pl.pallas_call(
        _xmlp_body,
        grid=grid,
        in_specs=[pl.BlockSpec((_R, 128), lambda i: (i, 0)),
                  full2((128, _HX)), full2((1, _HX)),
                  full2((_HX, _HX)), full2((1, _HX))],
        out_specs=hblk,
        out_shape=jax.ShapeDtypeStruct((2, _N, _HH), jnp.float32),
    )(X_t_one_hot, x_W1, vec(x_b1), x_W2, vec(x_b2))

    segsum = _make_segsum(epad)
    hs = [h]
    for l in range(3):
        aggr = segsum(h.reshape(_NC * _N, _HH), src2, dstr, wr)
        aggr = aggr.reshape(_NC, _N, _HH)
        h = pl.pallas_call(
            _layer_body,
            grid=grid,
            in_specs=[hblk, full3((2, _HH, _HX)), full2((1, _HX)),
                      full2((1, _HX)), full2((1, _HX))],
            out_specs=hblk,
            out_shape=jax.ShapeDtypeStruct((2, _N, _HH), jnp.float32),
        )(aggr, gws[l][:_HX].reshape(2, _HH, _HX), bgs[l],
          vec(lgs[l]), vec(lbs[l]))
        hs.append(h)

    out = pl.pallas_call(
        _out_body,
        grid=grid,
        in_specs=[hblk] * 4
        + [full3((8, _HH, _HCAT)), full2((1, _HCAT)),
           full2((_HCAT, 128)), full2((1, 128))],
        out_specs=pl.BlockSpec((_R, 128), lambda i: (i, 0)),
        out_shape=jax.ShapeDtypeStruct((_N, 128), jnp.float32),
    )(hs[0], hs[1], hs[2], hs[3],
      o_W1[:4 * _HX].reshape(8, _HH, _HCAT), bo, o_W2, vec(o_b2))
    return out


# speedup vs baseline: 5.7115x; 1.0239x over previous
import functools

import jax
import jax.numpy as jnp
from jax import lax
from jax.experimental import pallas as pl
from jax.experimental.pallas import tpu as pltpu
from jax.experimental.pallas import tpu_sc as plsc

_N = 10000
_HX = 256
_HH = 128
_HT = 128
_HCAT = 4 * _HX + _HT
_NC = 2
_NS = 16
_C = 112
_R = 400


def _prelude_body(t_ref, tw1, tb1, tw2, tb2, gt0, gb0, gt1, gb1, gt2, gb2,
                  owt, ob1, bg0, bg1, bg2, bo):
    x1 = jnp.maximum(t_ref[...] * tw1[...] + tb1[...], 0.0)
    ht = jnp.maximum(
        jnp.dot(x1, tw2[...], preferred_element_type=jnp.float32) + tb2[...],
        0.0)
    bg0[...] = jnp.dot(ht, gt0[...], preferred_element_type=jnp.float32) + gb0[...]
    bg1[...] = jnp.dot(ht, gt1[...], preferred_element_type=jnp.float32) + gb1[...]
    bg2[...] = jnp.dot(ht, gt2[...], preferred_element_type=jnp.float32) + gb2[...]
    bo[...] = jnp.dot(ht, owt[...], preferred_element_type=jnp.float32) + ob1[...]


def _xmlp_body(x, w1, b1, w2, b2, out):
    h1 = jnp.maximum(
        jnp.dot(x[...], w1[...], preferred_element_type=jnp.float32) + b1[...], 0.0)
    h = jnp.maximum(
        jnp.dot(h1, w2[...], preferred_element_type=jnp.float32) + b2[...], 0.0)
    out[0] = h[:, :_HH]
    out[1] = h[:, _HH:]


def _layer_body(a, w, bias, g, b, out):
    y = (jnp.dot(a[0], w[0], preferred_element_type=jnp.float32)
         + jnp.dot(a[1], w[1], preferred_element_type=jnp.float32)
         + bias[...])
    y = jnp.maximum(y, 0.0)
    m = jnp.mean(y, axis=-1, keepdims=True)
    yc = y - m
    v = jnp.mean(yc * yc, axis=-1, keepdims=True)
    h = yc * lax.rsqrt(v + 1e-5) * g[...] + b[...]
    out[0] = h[:, :_HH]
    out[1] = h[:, _HH:]


def _out_body(h0, h1, h2, h3, w1r, bo, w2, b2, out):
    s = None
    for k, h in enumerate((h0, h1, h2, h3)):
        for cc in range(2):
            contrib = jnp.dot(h[cc], w1r[2 * k + cc],
                              preferred_element_type=jnp.float32)
            s = contrib if s is None else s + contrib
    y = jnp.maximum(s + bo[...], 0.0)
    out[...] = jnp.dot(y, w2[...], preferred_element_type=jnp.float32) + b2[...]


_K = 6


@functools.lru_cache(maxsize=None)
def _make_segsum(epad):
    ept = epad // _NS
    nch = ept // _C
    nst = nch // _K
    assert nst % 2 == 0 and nst * _K * _NS * _C == epad
    nfull = _N // _C
    ntail = _N - nfull * _C

    mesh = plsc.VectorSubcoreMesh(core_axis_name="c", subcore_axis_name="s",
                                  num_cores=_NC, num_subcores=_NS)

    @functools.partial(
        pl.kernel,
        out_type=jax.ShapeDtypeStruct((_NC * _N, _HH), jnp.float32),
        mesh=mesh,
        scratch_types=[
            pltpu.VMEM((2, _K, _C), jnp.int32),
            pltpu.VMEM((2, _K, _C), jnp.int32),
            pltpu.VMEM((2, _K * _C), jnp.float32),
            pltpu.VMEM((_C, _HH), jnp.float32),
            pltpu.VMEM((_C, _HH), jnp.float32),
            pltpu.VMEM((_C, _HH), jnp.float32),
            pltpu.VMEM_SHARED((_N, _HH), jnp.float32),
            pltpu.SemaphoreType.DMA,
            pltpu.SemaphoreType.DMA,
            pltpu.SemaphoreType.DMA,
            pltpu.SemaphoreType.DMA,
            pltpu.SemaphoreType.DMA,
            pltpu.SemaphoreType.DMA,
            pltpu.SemaphoreType.DMA,
        ],
    )
    def segsum(hx, src2, dstr, wr, out, sidx, didx, wv, r0, r1, r2, aggr,
               g0, g1, g2, s0, s1, s2, isem):
        c = lax.axis_index("c")
        s = lax.axis_index("s")
        rowsl = (r0, r1, r2)
        gsems = (g0, g1, g2)
        ssems = (s0, s1, s2)
        widx = c * _NS + s
        z = jnp.zeros((16,), jnp.float32)

        @pl.loop(0, _C)
        def _(r):
            for j in range(_HH // 16):
                r0[r, pl.ds(16 * j, 16)] = z

        @pl.loop(s, nfull, step=_NS)
        def _(k):
            pltpu.sync_copy(r0, aggr.at[pl.ds(k * _C, _C)])

        @pl.when(s == 0)
        def _():
            pltpu.sync_copy(r0.at[pl.ds(0, ntail)],
                            aggr.at[pl.ds(nfull * _C, ntail)])

        plsc.subcore_barrier()

        def start_idx(pn, stn):
            pltpu.async_copy(src2.at[widx, stn], sidx.at[pn], isem)
            pltpu.async_copy(dstr.at[s, stn], didx.at[pn], isem)
            pltpu.async_copy(wr.at[s, stn], wv.at[pn], isem)

        def wait_idx(pn):
            pltpu.make_async_copy(src2.at[widx, 0], sidx.at[pn], isem).wait()
            pltpu.make_async_copy(dstr.at[s, 0], didx.at[pn], isem).wait()
            pltpu.make_async_copy(wr.at[s, 0], wv.at[pn], isem).wait()

        def start_gather(pn, ch, b):
            pltpu.async_copy(hx.at[sidx.at[pn, ch]], rowsl[b], gsems[b])

        def wait_gather(pn, ch, b):
            pltpu.make_async_copy(hx.at[sidx.at[pn, ch]], rowsl[b],
                                  gsems[b]).wait()

        def start_scatter(pn, ch, b):
            return pltpu.async_copy(rowsl[b], aggr.at[didx.at[pn, ch]],
                                    ssems[b], add=True)

        pltpu.sync_copy(src2.at[widx, 0], sidx.at[0])
        pltpu.sync_copy(dstr.at[s, 0], didx.at[0])
        pltpu.sync_copy(wr.at[s, 0], wv.at[0])
        start_gather(0, 0, 0)
        start_gather(0, 1, 1)

        @pl.loop(0, nst, step=2)
        def _(st0):
            scat = [None, None, None]
            for p in range(2):
                for ch in range(_K):
                    b = ch % 3
                    b2 = (b + 2) % 3
                    wait_gather(p, ch, b)
                    rows_b = rowsl[b]

                    @pl.loop(0, _C // 16)
                    def _(gg, p=p, ch=ch, rows_b=rows_b):
                        wvec = wv[p, pl.ds(ch * _C + gg * 16, 16)]
                        for ii in range(16):
                            wb = jnp.broadcast_to(wvec[ii], (16,))
                            for j in range(_HH // 16):
                                rows_b[gg * 16 + ii, pl.ds(16 * j, 16)] = (
                                    rows_b[gg * 16 + ii, pl.ds(16 * j, 16)]
                                    * wb)

                    if scat[b2] is not None:
                        scat[b2].wait()
                        scat[b2] = None
                    if ch == 1:
                        if p == 0:
                            start_idx(1, st0 + 1)
                        else:
                            @pl.when(st0 + 2 < nst)
                            def _():
                                start_idx(0, st0 + 2)
                    if ch < _K - 2:
                        start_gather(p, ch + 2, b2)
                    elif ch == _K - 2:
                        if p == 0:
                            wait_idx(1)
                            start_gather(1, 0, b2)
                        else:
                            @pl.when(st0 + 2 < nst)
                            def _():
                                wait_idx(0)
                                start_gather(0, 0, b2)
                    else:
                        if p == 0:
                            start_gather(1, 1, b2)
                        else:
                            @pl.when(st0 + 2 < nst)
                            def _():
                                start_gather(0, 1, b2)
                    if p == 1 and ch == _K - 1:
                        pltpu.sync_copy(rowsl[b], aggr.at[didx.at[p, ch]],
                                        add=True)
                    else:
                        scat[b] = start_scatter(p, ch, b)

        plsc.subcore_barrier()

        @pl.loop(s, nfull, step=_NS)
        def _(k):
            pltpu.sync_copy(aggr.at[pl.ds(k * _C, _C)],
                            out.at[pl.ds(c * _N + k * _C, _C)])

        @pl.when(s == 0)
        def _():
            pltpu.sync_copy(aggr.at[pl.ds(nfull * _C, ntail)],
                            out.at[pl.ds(c * _N + nfull * _C, ntail)])

    return segsum


def kernel(t_float, X_t_one_hot, edge_index, edge_weight, t_W1, t_b1, t_W2,
           t_b2, x_W1, x_b1, x_W2, x_b2, g_W0, g_b0, ln_g0, ln_b0, g_W1, g_b1,
           ln_g1, ln_b1, g_W2, g_b2, ln_g2, ln_b2, o_W1, o_b1, o_W2, o_b2):
    E = edge_index.shape[1]
    nst = -(-E // (_NS * _C * _K))
    nst += nst % 2
    epad = nst * _NS * _C * _K
    pad = epad - E
    dst = edge_index[0]
    src = edge_index[1]
    srcp = jnp.pad(src, (0, pad))
    dstp = jnp.pad(dst, (0, pad))
    wp = jnp.pad(edge_weight, (0, pad))
    src2 = jnp.stack([srcp, srcp + _N]).reshape(_NC * _NS, -1, _K, _C)
    dstr = dstp.reshape(_NS, -1, _K, _C)
    wr = wp.reshape(_NS, -1, _K * _C)

    gws = (g_W0, g_W1, g_W2)
    gbs = (g_b0, g_b1, g_b2)
    lgs = (ln_g0, ln_g1, ln_g2)
    lbs = (ln_b0, ln_b1, ln_b2)

    vec = lambda v: v.reshape(1, -1)
    bg0, bg1, bg2, bo = pl.pallas_call(
        _prelude_body,
        out_shape=[jax.ShapeDtypeStruct((1, _HX), jnp.float32)] * 3
        + [jax.ShapeDtypeStruct((1, _HCAT), jnp.float32)],
    )(vec(t_float), t_W1, vec(t_b1), t_W2, vec(t_b2),
      gws[0][_HX:], vec(gbs[0]), gws[1][_HX:], vec(gbs[1]),
      gws[2][_HX:], vec(gbs[2]), o_W1[4 * _HX:], vec(o_b1))
    bgs = (bg0, bg1, bg2)

    grid = (_N // _R,)
    full2 = lambda shape: pl.BlockSpec(shape, lambda i: (0, 0))
    full3 = lambda shape: pl.BlockSpec(shape, lambda i: (0, 0, 0))
    hblk = pl.BlockSpec((2, _R, _HH), lambda i: (0, i, 0))

    h = pl.pallas_call(
        _xmlp_body,
        grid=grid,
        in_specs=[pl.BlockSpec((_R, 128), lambda i: (i, 0)),
                  full2((128, _HX)), full2((1, _HX)),
                  full2((_HX, _HX)), full2((1, _HX))],
        out_specs=hblk,
        out_shape=jax.ShapeDtypeStruct((2, _N, _HH), jnp.float32),
    )(X_t_one_hot, x_W1, vec(x_b1), x_W2, vec(x_b2))

    segsum = _make_segsum(epad)
    hs = [h]
    for l in range(3):
        aggr = segsum(h.reshape(_NC * _N, _HH), src2, dstr, wr)
        aggr = aggr.reshape(_NC, _N, _HH)
        h = pl.pallas_call(
            _layer_body,
            grid=grid,
            in_specs=[hblk, full3((2, _HH, _HX)), full2((1, _HX)),
                      full2((1, _HX)), full2((1, _HX))],
            out_specs=hblk,
            out_shape=jax.ShapeDtypeStruct((2, _N, _HH), jnp.float32),
        )(aggr, gws[l][:_HX].reshape(2, _HH, _HX), bgs[l],
          vec(lgs[l]), vec(lbs[l]))
        hs.append(h)

    out = pl.pallas_call(
        _out_body,
        grid=grid,
        in_specs=[hblk] * 4
        + [full3((8, _HH, _HCAT)), full2((1, _HCAT)),
           full2((_HCAT, 128)), full2((1, 128))],
        out_specs=pl.BlockSpec((_R, 128), lambda i: (i, 0)),
        out_shape=jax.ShapeDtypeStruct((_N, 128), jnp.float32),
    )(hs[0], hs[1], hs[2], hs[3],
      o_W1[:4 * _HX].reshape(8, _HH, _HCAT), bo, o_W2, vec(o_b2))
    return out
